# baseline (speedup 1.0000x reference)
import jax
import jax.numpy as jnp
import numpy as np
from jax import lax
from jax.experimental import pallas as pl

_B, _S, _D = 1, 2048, 1024
_H, _KVH = 16, 4
_HD = _D // _H
_NREP = _H // _KVH
_ROPE_BASE = 10000.0
_TOPK = _S // 2

_BS = 256
_BQ = 512
_BO = 512

_SELSTEPS = 16

_CT = (((1,), (1,)), ((), ()))


def _rope(x, cos, sins):
    n = x.shape[1]
    lane = lax.broadcasted_iota(jnp.int32, (1, n), 1)
    shuf = jnp.where((lane % _HD) < (_HD // 2),
                     jnp.roll(x, -(_HD // 2), axis=1),
                     jnp.roll(x, _HD // 2, axis=1))
    return x * cos + shuf * sins


def _proj_kernel(h_ref, cos_ref, sins_ref,
                 wq_ref, bq_ref, wk_ref, bk_ref, wv_ref, bv_ref,
                 wr1_ref, br1_ref, wr2_ref, br2_ref,
                 q_out, k_out, v_out, r_out):
    h = h_ref[...]
    cos = jnp.tile(cos_ref[...], (1, _KVH))
    sins = jnp.tile(sins_ref[...], (1, _KVH))
    f32 = jnp.float32

    q1 = lax.dot_general(h, wq_ref[...], _CT, preferred_element_type=f32) + bq_ref[...]
    q_out[...] = _rope(q1, jnp.tile(cos, (1, _NREP)), jnp.tile(sins, (1, _NREP))) * 0.125

    k1 = lax.dot_general(h, wk_ref[...], _CT, preferred_element_type=f32) + bk_ref[...]
    kr = _rope(k1, cos, sins)
    vv = lax.dot_general(h, wv_ref[...], _CT, preferred_element_type=f32) + bv_ref[...]
    for g in range(_KVH):
        k_out[g] = kr[:, g * _HD:(g + 1) * _HD]
        v_out[g] = vv[:, g * _HD:(g + 1) * _HD]

    r1 = jnp.maximum(
        lax.dot_general(h, wr1_ref[...], _CT, preferred_element_type=f32) + br1_ref[...], 0.0)
    logits = lax.dot_general(r1, wr2_ref[...], _CT, preferred_element_type=f32) + br2_ref[...]
    m = jnp.max(logits, axis=1, keepdims=True)
    e = jnp.exp(logits - m)
    r_out[...] = jnp.swapaxes(e / jnp.sum(e, axis=1, keepdims=True), 0, 1)


def _attn_kernel(q_ref, k_ref, v_ref, r_ref, o_ref):
    f32 = jnp.float32
    q2h = q_ref[...]
    k = k_ref[0]
    v = v_ref[0]
    v_ext = jnp.concatenate([v, jnp.ones((_S, 1), f32)], axis=1)
    kf = np.float32(_TOPK)

    s = [lax.dot_general(q2h[:, t * _HD:(t + 1) * _HD], k, _CT,
                         preferred_element_type=f32) for t in range(2)]

    m = [jnp.max(st, axis=1, keepdims=True) for st in s]
    lo = [jnp.min(st, axis=1, keepdims=True) for st in s]
    hi = list(m)
    for _ in range(_SELSTEPS):
        mid = [0.5 * (lo[t] + hi[t]) for t in range(2)]
        cnt = [jnp.sum(jnp.where(s[t] >= mid[t], 1.0, 0.0), axis=1,
                       keepdims=True) for t in range(2)]
        for t in range(2):
            ok = cnt[t] >= kf
            lo[t] = jnp.where(ok, mid[t], lo[t])
            hi[t] = jnp.where(ok, hi[t], mid[t])

    outs = []
    for t in range(2):
        p = jnp.where(s[t] >= lo[t], jnp.exp(s[t] - m[t]), 0.0)
        oe = lax.dot_general(p, v_ext, (((1,), (0,)), ((), ())),
                             preferred_element_type=f32)
        scale = r_ref[t, 0, 0].reshape(_BQ, 1) / oe[:, _HD:_HD + 1]
        outs.append(oe[:, :_HD] * scale)
    o_ref[...] = jnp.concatenate(outs, axis=1)


def _oproj_kernel(a_ref, wo_ref, bo_ref, o_ref):
    o_ref[...] = lax.dot_general(a_ref[...], wo_ref[...], _CT,
                                 preferred_element_type=jnp.float32) + bo_ref[...]


def kernel(hidden_states, Wq, bq, Wk, bk, Wv, bv, Wo, bo, Wr1, br1, Wr2, br2):
    f32 = jnp.float32
    h2 = hidden_states.reshape(_S, _D)

    pos = jnp.arange(_S, dtype=f32)
    inv_freq = 1.0 / (_ROPE_BASE ** (jnp.arange(0, _HD, 2, dtype=f32) / _HD))
    freqs = pos[:, None] * inv_freq[None, :]
    emb = jnp.concatenate((freqs, freqs), axis=-1)
    sin_sgn = jnp.concatenate(
        (-jnp.sin(emb[:, : _HD // 2]), jnp.sin(emb[:, _HD // 2:])), axis=1)
    cos_t = jnp.cos(emb)
    sins_t = sin_sgn

    row2 = lambda x: x.reshape(1, -1)

    q, k, v, r = pl.pallas_call(
        _proj_kernel,
        grid=(_S // _BS,),
        in_specs=[
            pl.BlockSpec((_BS, _D), lambda i: (i, 0)),
            pl.BlockSpec((_BS, _HD), lambda i: (i, 0)),
            pl.BlockSpec((_BS, _HD), lambda i: (i, 0)),
            pl.BlockSpec((_H * _HD, _D), lambda i: (0, 0)),
            pl.BlockSpec((1, _H * _HD), lambda i: (0, 0)),
            pl.BlockSpec((_KVH * _HD, _D), lambda i: (0, 0)),
            pl.BlockSpec((1, _KVH * _HD), lambda i: (0, 0)),
            pl.BlockSpec((_KVH * _HD, _D), lambda i: (0, 0)),
            pl.BlockSpec((1, _KVH * _HD), lambda i: (0, 0)),
            pl.BlockSpec((_D // 2, _D), lambda i: (0, 0)),
            pl.BlockSpec((1, _D // 2), lambda i: (0, 0)),
            pl.BlockSpec((_H, _D // 2), lambda i: (0, 0)),
            pl.BlockSpec((1, _H), lambda i: (0, 0)),
        ],
        out_specs=[
            pl.BlockSpec((_BS, _H * _HD), lambda i: (i, 0)),
            pl.BlockSpec((_KVH, _BS, _HD), lambda i: (0, i, 0)),
            pl.BlockSpec((_KVH, _BS, _HD), lambda i: (0, i, 0)),
            pl.BlockSpec((_H, _BS), lambda i: (0, i)),
        ],
        out_shape=[
            jax.ShapeDtypeStruct((_S, _H * _HD), f32),
            jax.ShapeDtypeStruct((_KVH, _S, _HD), f32),
            jax.ShapeDtypeStruct((_KVH, _S, _HD), f32),
            jax.ShapeDtypeStruct((_H, _S), f32),
        ],
    )(h2, cos_t, sins_t,
      Wq, row2(bq), Wk, row2(bk), Wv, row2(bv),
      Wr1, row2(br1), Wr2, row2(br2))

    k4, v4 = k, v
    r4 = r.reshape(_H, _S // _BQ, 1, _BQ)

    a2 = pl.pallas_call(
        _attn_kernel,
        grid=(_H // 2, _S // _BQ),
        in_specs=[
            pl.BlockSpec((_BQ, 2 * _HD), lambda hp, i: (i, hp)),
            pl.BlockSpec((1, _S, _HD), lambda hp, i: (hp // 2, 0, 0)),
            pl.BlockSpec((1, _S, _HD), lambda hp, i: (hp // 2, 0, 0)),
            pl.BlockSpec((2, 1, 1, _BQ), lambda hp, i: (hp, i, 0, 0)),
        ],
        out_specs=pl.BlockSpec((_BQ, 2 * _HD), lambda hp, i: (i, hp)),
        out_shape=jax.ShapeDtypeStruct((_S, _H * _HD), f32),
    )(q, k4, v4, r4)

    out = pl.pallas_call(
        _oproj_kernel,
        grid=(_S // _BO,),
        in_specs=[
            pl.BlockSpec((_BO, _H * _HD), lambda i: (i, 0)),
            pl.BlockSpec((_D, _H * _HD), lambda i: (0, 0)),
            pl.BlockSpec((1, _D), lambda i: (0, 0)),
        ],
        out_specs=pl.BlockSpec((_BO, _D), lambda i: (i, 0)),
        out_shape=jax.ShapeDtypeStruct((_S, _D), f32),
    )(a2, Wo, row2(bo))

    return out.reshape(_B, _S, _D)
